# Initial kernel scaffold; baseline (speedup 1.0000x reference)
import jax
import jax.numpy as jnp
from jax.experimental import pallas as pl


def kernel(graph_embedding, edge_index, e_feat, weight):
    raise NotImplementedError("write your pallas kernel here")


# speedup vs baseline: 4.8562x; 4.8562x over previous
import functools

import jax
import jax.numpy as jnp
from jax import lax
from jax.experimental import pallas as pl
from jax.experimental.pallas import tpu as pltpu
from jax.experimental.pallas import tpu_sc as plsc

N = 10000
E = 320000
D = 128

NC = 2
NS = 16
L = 16

H = N // NC
H_PAD = 5008
ACC_ROWS = H_PAD + NS
EPT = E // NS
SUB = 4000
NSUB = EPT // SUB
G = 80
NGRP = SUB // G

RPT = 312
LAST_ROWS = H - 15 * RPT
ZPT = 312
LAST_ZROWS = ACC_ROWS - 15 * ZPT


def _scale_store_rows(rows_v, wv, n_rows):
  def body(i, _):
    for j in range(D // L):
      sl = pl.ds(j * L, L)
      rows_v[i, sl] = rows_v[i, sl] * wv[sl]
    return 0

  lax.fori_loop(0, n_rows, body, 0)


def _sc_kernel(ge_hbm, src_hbm, dst_hbm, ef_hbm, w_hbm,
               out1_hbm, out0_hbm,
               src_v, dst_v, ef_v, rows_v, idx1_v, idx0_v, wv, sem,
               acc1, acc0):
  c = lax.axis_index("c")
  s = lax.axis_index("s")
  lo = c * H
  dump = H_PAD + s
  ebase = s * EPT

  pltpu.sync_copy(w_hbm, wv)

  zero = jnp.zeros((L,), jnp.float32)

  def zrow(i, _):
    for j in range(D // L):
      rows_v[i, pl.ds(j * L, L)] = zero
    return 0

  lax.fori_loop(0, G, zrow, 0)

  def zero_acc_slice(n_rows):
    zbase = s * ZPT
    for acc in (acc1, acc0):
      off = 0
      while off < n_rows:
        ck = min(G, n_rows - off)
        pltpu.sync_copy(rows_v.at[pl.ds(0, ck)], acc.at[pl.ds(zbase + off, ck)])
        off += ck

  @pl.when(s < NS - 1)
  def _():
    zero_acc_slice(ZPT)

  @pl.when(s == NS - 1)
  def _():
    zero_acc_slice(LAST_ZROWS)

  plsc.subcore_barrier()

  def sub(sc, _):
    soff = ebase + sc * SUB
    pltpu.sync_copy(src_hbm.at[pl.ds(soff, SUB)], src_v)
    pltpu.sync_copy(dst_hbm.at[pl.ds(soff, SUB)], dst_v)
    pltpu.sync_copy(ef_hbm.at[pl.ds(soff, SUB)], ef_v)

    def grp(g, _):
      eb = g * G
      for j in range(G // L):
        sl = pl.ds(eb + j * L, L)
        dv = dst_v[sl]
        ev = ef_v[sl]
        inr = (dv >= lo) & (dv < lo + H)
        dl = dv - lo
        dumpvec = jnp.zeros((L,), jnp.int32) + dump
        idx1_v[pl.ds(j * L, L)] = jnp.where(inr, dl, dumpvec)
        idx0_v[pl.ds(j * L, L)] = jnp.where(inr & (ev == 0), dl, dumpvec)
      pltpu.async_copy(ge_hbm.at[src_v.at[pl.ds(eb, G)]], rows_v, sem).wait()
      pltpu.sync_copy(rows_v, acc1.at[idx1_v], add=True)
      pltpu.sync_copy(rows_v, acc0.at[idx0_v], add=True)
      return 0

    lax.fori_loop(0, NGRP, grp, 0)
    return 0

  lax.fori_loop(0, NSUB, sub, 0)
  plsc.subcore_barrier()

  def flush(acc, out_hbm, n_rows):
    obase = lo + s * RPT
    off = 0
    while off < n_rows:
      ck = min(G, n_rows - off)
      pltpu.sync_copy(acc.at[pl.ds(s * RPT + off, ck)], rows_v.at[pl.ds(0, ck)])
      _scale_store_rows(rows_v, wv, ck)
      pltpu.sync_copy(rows_v.at[pl.ds(0, ck)], out_hbm.at[pl.ds(obase + off, ck)])
      off += ck

  @pl.when(s < NS - 1)
  def _():
    flush(acc1, out1_hbm, RPT)
    flush(acc0, out0_hbm, RPT)

  @pl.when(s == NS - 1)
  def _():
    flush(acc1, out1_hbm, LAST_ROWS)
    flush(acc0, out0_hbm, LAST_ROWS)


@jax.jit
def _run(graph_embedding, src, dst, e_feat, w):
  mesh = plsc.VectorSubcoreMesh(core_axis_name="c", subcore_axis_name="s")
  f = pl.kernel(
      _sc_kernel,
      out_type=(
          jax.ShapeDtypeStruct((N, D), jnp.float32),
          jax.ShapeDtypeStruct((N, D), jnp.float32),
      ),
      mesh=mesh,
      scratch_types=[
          pltpu.VMEM((SUB,), jnp.int32),
          pltpu.VMEM((SUB,), jnp.int32),
          pltpu.VMEM((SUB,), jnp.int32),
          pltpu.VMEM((G, D), jnp.float32),
          pltpu.VMEM((G,), jnp.int32),
          pltpu.VMEM((G,), jnp.int32),
          pltpu.VMEM((D,), jnp.float32),
          pltpu.SemaphoreType.DMA,
          pltpu.VMEM_SHARED((ACC_ROWS, D), jnp.float32),
          pltpu.VMEM_SHARED((ACC_ROWS, D), jnp.float32),
      ],
  )
  return f(graph_embedding, src, dst, e_feat, w)


def kernel(graph_embedding, edge_index, e_feat, weight):
  src = edge_index[0]
  dst = edge_index[1]
  w = weight.reshape(D)
  res, res0 = _run(graph_embedding, src, dst, e_feat, w)
  return (res, res0, res0, res0, res0)


# speedup vs baseline: 5.8913x; 1.2132x over previous
import functools

import jax
import jax.numpy as jnp
from jax import lax
from jax.experimental import pallas as pl
from jax.experimental.pallas import tpu as pltpu
from jax.experimental.pallas import tpu_sc as plsc

N = 10000
E = 320000
D = 128

NC = 2
NS = 16
L = 16

H = N // NC
H_PAD = 5008
ACC_ROWS = H_PAD + NS
EPT = E // NS
SUB = 4000
NSUB = EPT // SUB
G = 80
NGRP = SUB // G

RPT = 312
LAST_ROWS = H - 15 * RPT
ZPT = 312
LAST_ZROWS = ACC_ROWS - 15 * ZPT


def _scale_store_rows(rows_v, wv, n_rows):
  def body(i, _):
    for j in range(D // L):
      sl = pl.ds(j * L, L)
      rows_v[0, i, sl] = rows_v[0, i, sl] * wv[sl]
    return 0

  lax.fori_loop(0, n_rows, body, 0)


def _sc_kernel(ge_hbm, src_hbm, dst_hbm, ef_hbm, w_hbm,
               out1_hbm, out0_hbm,
               src_v, dst_v, ef_v, rows_v, idx1_v, idx0_v, wv,
               gsem0, gsem1, ssem0, ssem1,
               acc1, acc0):
  c = lax.axis_index("c")
  s = lax.axis_index("s")
  lo = c * H
  dump = H_PAD + s
  ebase = s * EPT

  pltpu.sync_copy(w_hbm, wv)

  zero = jnp.zeros((L,), jnp.float32)

  def zrow(i, _):
    for j in range(D // L):
      rows_v[0, i, pl.ds(j * L, L)] = zero
    return 0

  lax.fori_loop(0, G, zrow, 0)

  def zero_acc_slice(n_rows):
    zbase = s * ZPT
    for acc in (acc1, acc0):
      off = 0
      while off < n_rows:
        ck = min(G, n_rows - off)
        pltpu.sync_copy(rows_v.at[0, pl.ds(0, ck)], acc.at[pl.ds(zbase + off, ck)])
        off += ck

  @pl.when(s < NS - 1)
  def _():
    zero_acc_slice(ZPT)

  @pl.when(s == NS - 1)
  def _():
    zero_acc_slice(LAST_ZROWS)

  plsc.subcore_barrier()

  def sub(sc, _):
    soff = ebase + sc * SUB
    pltpu.sync_copy(src_hbm.at[pl.ds(soff, SUB)], src_v)
    pltpu.sync_copy(dst_hbm.at[pl.ds(soff, SUB)], dst_v)
    pltpu.sync_copy(ef_hbm.at[pl.ds(soff, SUB)], ef_v)

    dumpvec = jnp.zeros((L,), jnp.int32) + dump

    def compute_idx(eb, b):
      for j in range(G // L):
        sl = pl.ds(eb + j * L, L)
        dv = dst_v[sl]
        ev = ef_v[sl]
        inr = (dv >= lo) & (dv < lo + H)
        dl = dv - lo
        idx1_v[b, pl.ds(j * L, L)] = jnp.where(inr, dl, dumpvec)
        idx0_v[b, pl.ds(j * L, L)] = jnp.where(inr & (ev == 0), dl, dumpvec)

    def pair(p, _):
      e0 = p * (2 * G)
      e1 = e0 + G
      compute_idx(e0, 0)
      d0 = pltpu.async_copy(
          ge_hbm.at[src_v.at[pl.ds(e0, G)]], rows_v.at[0], gsem0)
      compute_idx(e1, 1)
      d1 = pltpu.async_copy(
          ge_hbm.at[src_v.at[pl.ds(e1, G)]], rows_v.at[1], gsem1)
      d0.wait()
      s0a = pltpu.async_copy(rows_v.at[0], acc1.at[idx1_v.at[0]], ssem0,
                             add=True)
      s0b = pltpu.async_copy(rows_v.at[0], acc0.at[idx0_v.at[0]], ssem0,
                             add=True)
      d1.wait()
      s1a = pltpu.async_copy(rows_v.at[1], acc1.at[idx1_v.at[1]], ssem1,
                             add=True)
      s1b = pltpu.async_copy(rows_v.at[1], acc0.at[idx0_v.at[1]], ssem1,
                             add=True)
      s0a.wait()
      s0b.wait()
      s1a.wait()
      s1b.wait()
      return 0

    lax.fori_loop(0, NGRP // 2, pair, 0)
    return 0

  lax.fori_loop(0, NSUB, sub, 0)
  plsc.subcore_barrier()

  def flush(acc, out_hbm, n_rows):
    obase = lo + s * RPT
    off = 0
    while off < n_rows:
      ck = min(G, n_rows - off)
      pltpu.sync_copy(acc.at[pl.ds(s * RPT + off, ck)],
                      rows_v.at[0, pl.ds(0, ck)])
      _scale_store_rows(rows_v, wv, ck)
      pltpu.sync_copy(rows_v.at[0, pl.ds(0, ck)],
                      out_hbm.at[pl.ds(obase + off, ck)])
      off += ck

  @pl.when(s < NS - 1)
  def _():
    flush(acc1, out1_hbm, RPT)
    flush(acc0, out0_hbm, RPT)

  @pl.when(s == NS - 1)
  def _():
    flush(acc1, out1_hbm, LAST_ROWS)
    flush(acc0, out0_hbm, LAST_ROWS)


@jax.jit
def _run(graph_embedding, src, dst, e_feat, w):
  mesh = plsc.VectorSubcoreMesh(core_axis_name="c", subcore_axis_name="s")
  f = pl.kernel(
      _sc_kernel,
      out_type=(
          jax.ShapeDtypeStruct((N, D), jnp.float32),
          jax.ShapeDtypeStruct((N, D), jnp.float32),
      ),
      mesh=mesh,
      scratch_types=[
          pltpu.VMEM((SUB,), jnp.int32),
          pltpu.VMEM((SUB,), jnp.int32),
          pltpu.VMEM((SUB,), jnp.int32),
          pltpu.VMEM((2, G, D), jnp.float32),
          pltpu.VMEM((2, G), jnp.int32),
          pltpu.VMEM((2, G), jnp.int32),
          pltpu.VMEM((D,), jnp.float32),
          pltpu.SemaphoreType.DMA,
          pltpu.SemaphoreType.DMA,
          pltpu.SemaphoreType.DMA,
          pltpu.SemaphoreType.DMA,
          pltpu.VMEM_SHARED((ACC_ROWS, D), jnp.float32),
          pltpu.VMEM_SHARED((ACC_ROWS, D), jnp.float32),
      ],
  )
  return f(graph_embedding, src, dst, e_feat, w)


def kernel(graph_embedding, edge_index, e_feat, weight):
  src = edge_index[0]
  dst = edge_index[1]
  w = weight.reshape(D)
  res, res0 = _run(graph_embedding, src, dst, e_feat, w)
  return (res, res0, res0, res0, res0)
